# baseline (speedup 1.0000x reference)
import functools

import jax
import jax.numpy as jnp
from jax import lax
from jax.experimental import pallas as pl
from jax.experimental.pallas import tpu as pltpu
from jax.experimental.pallas import tpu_sc as plsc

M = 1_000_000
D = 64
NC = 2
NS = 16
NW = NC * NS
C = 800
CHUNK_F = C * D
N_CHUNKS = M // C
ITERS = (N_CHUNKS + NW - 1) // NW
GROUPS = C // 16


def _body(items_hbm, u_hbm, out_hbm, in_buf0, in_buf1, out_buf, u_vmem,
          sem0, sem1):
    wid = lax.axis_index("s") * NC + lax.axis_index("c")
    in_bufs = (in_buf0, in_buf1)
    sems = (sem0, sem1)

    pltpu.sync_copy(u_hbm, u_vmem)

    lanes = lax.iota(jnp.int32, 16)

    def chunk_off(j):
        return (wid + NW * j) * CHUNK_F

    def start_in(j, b):
        pltpu.async_copy(
            items_hbm.at[pl.ds(chunk_off(j), CHUNK_F)], in_bufs[b], sems[b])

    def wait_in(j, b):
        pltpu.make_async_copy(
            items_hbm.at[pl.ds(chunk_off(j), CHUNK_F)], in_bufs[b],
            sems[b]).wait()

    def compute(j, b):
        buf = in_bufs[b]

        def group(g, _):
            idx0 = g * (16 * D) + lanes * D
            acc = jnp.zeros((16,), jnp.float32)
            for d in range(D):
                v = plsc.load_gather(buf, [idx0 + d])
                acc = acc + v * u_vmem[d, :]
            out_buf[pl.ds(g * 16, 16)] = acc
            return 0

        lax.fori_loop(0, GROUPS, group, 0)
        pltpu.sync_copy(out_buf, out_hbm.at[pl.ds((wid + NW * j) * C, C)])

    start_in(0, 0)

    def step(jp, _):
        for b in (0, 1):
            j = 2 * jp + b
            nxt = j + 1
            nxt_valid = jnp.logical_and(nxt < ITERS,
                                        wid + NW * nxt < N_CHUNKS)
            cur_valid = wid + NW * j < N_CHUNKS

            @pl.when(nxt_valid)
            def _():
                start_in(nxt, 1 - b)

            @pl.when(cur_valid)
            def _():
                wait_in(j, b)
                compute(j, b)
        return 0

    lax.fori_loop(0, ITERS // 2, step, 0)


@jax.jit
def _sc_matvec(items_flat, u_b):
    mesh = plsc.VectorSubcoreMesh(core_axis_name="c", subcore_axis_name="s")
    f = pl.kernel(
        _body,
        out_type=jax.ShapeDtypeStruct((M,), jnp.float32),
        mesh=mesh,
        scratch_types=[
            pltpu.VMEM((CHUNK_F,), jnp.float32),
            pltpu.VMEM((CHUNK_F,), jnp.float32),
            pltpu.VMEM((C,), jnp.float32),
            pltpu.VMEM((D, 16), jnp.float32),
            pltpu.SemaphoreType.DMA,
            pltpu.SemaphoreType.DMA,
        ],
        compiler_params=pltpu.CompilerParams(needs_layout_passes=False),
    )
    return f(items_flat, u_b)


def kernel(items_emb, user_emb):
    items_flat = items_emb.reshape(-1)
    u_b = jnp.broadcast_to(user_emb.reshape(D, 1), (D, 16))
    return _sc_matvec(items_flat, u_b)
